# baseline (speedup 1.0000x reference)
import functools

import jax
import jax.numpy as jnp
from jax import lax
from jax.experimental import pallas as pl
from jax.experimental.pallas import tpu as pltpu
from jax.experimental.pallas import tpu_sc as plsc

CB = 1024
DP = 128


def _vq_body(zsq_ref, z_ref, w_ref, idx_ref, loss_ref):
    zb = z_ref[0]
    w = w_ref[...]
    t = zb.shape[1]
    s2 = jax.lax.dot_general((-2.0 * w).astype(jnp.bfloat16),
                             zb.astype(jnp.bfloat16),
                             (((1,), (0,)), ((), ())),
                             preferred_element_type=jnp.float32)
    zsq = zsq_ref[0]
    wsq = jnp.sum(w * w, axis=1, keepdims=True)
    d2 = (zsq + wsq) + s2
    dist = jnp.sqrt(jnp.maximum(d2, 0.0))
    m = jnp.min(dist, axis=0, keepdims=True)
    iota_f = jax.lax.broadcasted_iota(jnp.int32, (CB, t), 0).astype(jnp.float32)
    idxf = jnp.min(jnp.where(dist == m, iota_f, float(CB)), axis=0)
    idx_ref[0, 0, :] = idxf.astype(jnp.int32)
    loss_ref[0, 0, :] = (m * m)[0]


def _make_gather(n_tok):
    info = plsc.get_sparse_core_info()
    nw = info.num_cores * info.num_subcores
    b_per_w = n_tok // nw
    chunk = min(b_per_w, 512)
    mesh = plsc.VectorSubcoreMesh(core_axis_name="c", subcore_axis_name="s")

    @functools.partial(
        pl.kernel, mesh=mesh,
        out_type=jax.ShapeDtypeStruct((n_tok, DP), jnp.float32),
        scratch_types=[
            pltpu.VMEM((chunk,), jnp.int32),
            pltpu.VMEM((chunk, DP), jnp.float32),
            pltpu.SemaphoreType.DMA,
        ],
    )
    def gather_k(table_hbm, idx_hbm, out_hbm, idx_v, rows_v, sem):
        wid = lax.axis_index("s") * info.num_cores + lax.axis_index("c")
        for c in range(b_per_w // chunk):
            base = wid * b_per_w + c * chunk
            pltpu.sync_copy(idx_hbm.at[pl.ds(base, chunk)], idx_v)
            pltpu.async_copy(table_hbm.at[idx_v], rows_v, sem).wait()
            pltpu.sync_copy(rows_v, out_hbm.at[pl.ds(base, chunk)])

    return gather_k


def _tc_half(zsq, z, W, d, T):
    B = z.shape[0]
    tb = 2048
    return pl.pallas_call(
        _vq_body,
        grid=(B, T // tb),
        in_specs=[
            pl.BlockSpec((1, 1, tb), lambda i, j: (i, 0, j)),
            pl.BlockSpec((1, d, tb), lambda i, j: (i, 0, j)),
            pl.BlockSpec((CB, d), lambda i, j: (0, 0)),
        ],
        out_specs=[
            pl.BlockSpec((1, 1, tb), lambda i, j: (i, 0, j)),
            pl.BlockSpec((1, 1, tb), lambda i, j: (i, 0, j)),
        ],
        out_shape=[
            jax.ShapeDtypeStruct((B, 1, T), jnp.int32),
            jax.ShapeDtypeStruct((B, 1, T), jnp.float32),
        ],
        compiler_params=pltpu.CompilerParams(
            dimension_semantics=("parallel", "parallel"),
        ),
    )(zsq, z, W)


@jax.jit
def kernel(z, W):
    B, d, T = z.shape
    z_flat = jnp.transpose(z, (0, 2, 1)).reshape(-1, d)
    zsq = jnp.sum(z_flat * z_flat, axis=1).reshape(B, 1, T)
    w_pad = jnp.pad(W, ((0, 0), (0, DP - d)))
    hb = B // 2
    gather = _make_gather(hb * T)
    idx_a, loss_a = _tc_half(zsq[:hb], z[:hb], W, d, T)
    idx_b, loss_b = _tc_half(zsq[hb:], z[hb:], W, d, T)
    q_a = gather(w_pad, idx_a.reshape(-1))
    q_b = gather(w_pad, idx_b.reshape(-1))
    indices = jnp.concatenate([idx_a, idx_b], axis=0).reshape(B, T)
    q_pad = jnp.concatenate([q_a, q_b], axis=0)
    quantized = jnp.transpose(q_pad[:, :d].reshape(B, T, d), (0, 2, 1))
    commit_loss = (jnp.sum(loss_a) + jnp.sum(loss_b)) / (B * d * T)
    return (quantized, indices, commit_loss)


# speedup vs baseline: 1.1449x; 1.1449x over previous
import jax
import jax.numpy as jnp
from jax.experimental import pallas as pl
from jax.experimental.pallas import tpu as pltpu

CB = 1024


def _vq_body(zsq_ref, z_ref, w_ref, q_ref, idx_ref, loss_ref):
    zb = z_ref[0]
    w = w_ref[...]
    t = zb.shape[1]
    s2 = jax.lax.dot_general((-2.0 * w).astype(jnp.bfloat16),
                             zb.astype(jnp.bfloat16),
                             (((1,), (0,)), ((), ())),
                             preferred_element_type=jnp.float32)
    zsq = zsq_ref[0]
    wsq = jnp.sum(w * w, axis=1, keepdims=True)
    d2 = (zsq + wsq) + s2
    dist = jnp.sqrt(jnp.maximum(d2, 0.0))
    m = jnp.min(dist, axis=0, keepdims=True)
    iota_f = jax.lax.broadcasted_iota(jnp.int32, (CB, t), 0).astype(jnp.float32)
    idxf = jnp.min(jnp.where(dist == m, iota_f, float(CB)), axis=0)
    idx = idxf.astype(jnp.int32)
    idx_ref[0, 0, :] = idx
    onehot = (iota_f == idxf[None, :]).astype(jnp.bfloat16)
    w_hi = w.astype(jnp.bfloat16)
    r1 = w - w_hi.astype(jnp.float32)
    w_mid = r1.astype(jnp.bfloat16)
    w_lo = (r1 - w_mid.astype(jnp.float32)).astype(jnp.bfloat16)
    dn = (((0,), (0,)), ((), ()))
    q_hi = jax.lax.dot_general(w_hi, onehot, dn, preferred_element_type=jnp.float32)
    q_mid = jax.lax.dot_general(w_mid, onehot, dn, preferred_element_type=jnp.float32)
    q_lo = jax.lax.dot_general(w_lo, onehot, dn, preferred_element_type=jnp.float32)
    q = (q_hi + q_mid) + q_lo
    q_ref[0] = q
    loss_ref[0, 0, :] = jnp.sum((zb - q) ** 2, axis=0)


@jax.jit
def kernel(z, W):
    B, d, T = z.shape
    z_flat = jnp.transpose(z, (0, 2, 1)).reshape(-1, d)
    zsq = jnp.sum(z_flat * z_flat, axis=1).reshape(B, 1, T)
    tb = 2048
    q, idx3, lossp = pl.pallas_call(
        _vq_body,
        grid=(B, T // tb),
        in_specs=[
            pl.BlockSpec((1, 1, tb), lambda i, j: (i, 0, j)),
            pl.BlockSpec((1, d, tb), lambda i, j: (i, 0, j)),
            pl.BlockSpec((CB, d), lambda i, j: (0, 0)),
        ],
        out_specs=[
            pl.BlockSpec((1, d, tb), lambda i, j: (i, 0, j)),
            pl.BlockSpec((1, 1, tb), lambda i, j: (i, 0, j)),
            pl.BlockSpec((1, 1, tb), lambda i, j: (i, 0, j)),
        ],
        out_shape=[
            jax.ShapeDtypeStruct((B, d, T), jnp.float32),
            jax.ShapeDtypeStruct((B, 1, T), jnp.int32),
            jax.ShapeDtypeStruct((B, 1, T), jnp.float32),
        ],
        compiler_params=pltpu.CompilerParams(
            dimension_semantics=("parallel", "parallel"),
        ),
    )(zsq, z, W)
    indices = idx3.reshape(B, T)
    commit_loss = jnp.sum(lossp) / (B * d * T)
    return (q, indices, commit_loss)
